# baseline (speedup 1.0000x reference)
import dataclasses
import functools

import jax
import jax.numpy as jnp
from jax import lax
from jax.experimental import pallas as pl
from jax.experimental.pallas import tpu as pltpu
from jax.experimental.pallas import tpu_sc as plsc

NC = 2
NS = 16
NW = NC * NS
LANES = 16
K = 128
ZR = 16


def _stage_size(cpt):
  for s in (16, 8):
    if cpt % s == 0:
      return s
  return cpt


def _sc_mesh():
  return plsc.VectorSubcoreMesh(core_axis_name="c", subcore_axis_name="s")


def _sc_params():
  cp = pltpu.CompilerParams()
  if "needs_layout_passes" in pltpu.CompilerParams.__dataclass_fields__:
    cp = dataclasses.replace(cp, needs_layout_passes=False)
  return cp


def _make_deg_kernel(n_pad, cpt):
  @functools.partial(
      pl.kernel,
      out_type=jax.ShapeDtypeStruct((NW, n_pad), jnp.float32),
      mesh=_sc_mesh(),
      scratch_types=[
          pltpu.VMEM((n_pad,), jnp.float32),
          pltpu.VMEM((cpt, K), jnp.int32),
      ],
      compiler_params=_sc_params(),
  )
  def deg_kernel(dst_hbm, out_hbm, deg_v, idx_v):
    c = lax.axis_index("c")
    s = lax.axis_index("s")
    wid = s * NC + c
    zeros16 = jnp.zeros((LANES,), jnp.float32)
    ones16 = jnp.ones((LANES,), jnp.float32)

    @pl.loop(0, n_pad // LANES)
    def _(i):
      deg_v[pl.ds(i * LANES, LANES)] = zeros16

    pltpu.sync_copy(dst_hbm.at[wid], idx_v)

    @pl.loop(0, cpt)
    def _(j):
      for q in range(K // LANES):
        idx = idx_v[j, pl.ds(q * LANES, LANES)]
        plsc.addupdate_scatter(deg_v, [idx], ones16)

    pltpu.sync_copy(deg_v, out_hbm.at[wid])

  return deg_kernel


def _make_agg_kernel(n_pad, cpt, dh):
  rps = n_pad // NS
  sg = _stage_size(cpt)
  nstage = cpt // sg

  @functools.partial(
      pl.kernel,
      out_type=jax.ShapeDtypeStruct((NC, n_pad, dh), jnp.float32),
      mesh=_sc_mesh(),
      scratch_types=[
          pltpu.VMEM((sg, K), jnp.int32),
          pltpu.VMEM((sg, K), jnp.int32),
          pltpu.VMEM((K, dh), jnp.float32),
          pltpu.VMEM((K, dh), jnp.float32),
          pltpu.VMEM((ZR, dh), jnp.float32),
          pltpu.VMEM_SHARED((n_pad, dh), jnp.float32),
          pltpu.SemaphoreType.DMA,
          pltpu.SemaphoreType.DMA,
      ],
      compiler_params=_sc_params(),
  )
  def agg_kernel(y_hbm, src_hbm, dst_hbm, out_hbm,
                 src_v, dst_v, rows0_v, rows1_v, zero_v, accum_sh,
                 gsem0, gsem1):
    c = lax.axis_index("c")
    s = lax.axis_index("s")
    wid = s * NC + c
    zeros16 = jnp.zeros((LANES,), jnp.float32)

    def start_g(j, rows_v, sem):
      pltpu.make_async_copy(y_hbm.at[src_v.at[j]], rows_v, sem).start()

    def wait_g(rows_v, sem):
      pltpu.make_async_copy(y_hbm.at[src_v.at[0]], rows_v, sem).wait()

    def scat(j, rows_v):
      pltpu.sync_copy(rows_v, accum_sh.at[dst_v.at[j]], add=True)

    @pl.loop(0, ZR)
    def _(r):
      for q in range(dh // LANES):
        zero_v[r, pl.ds(q * LANES, LANES)] = zeros16

    @pl.loop(0, rps // ZR)
    def _(z):
      pltpu.sync_copy(zero_v, accum_sh.at[pl.ds(s * rps + z * ZR, ZR)])

    plsc.subcore_barrier()

    @pl.loop(0, nstage)
    def _(st):
      pltpu.sync_copy(src_hbm.at[wid].at[pl.ds(st * sg, sg)], src_v)
      pltpu.sync_copy(dst_hbm.at[wid].at[pl.ds(st * sg, sg)], dst_v)
      start_g(0, rows0_v, gsem0)

      @pl.loop(0, sg - 2, step=2)
      def _(jj):
        wait_g(rows0_v, gsem0)
        start_g(jj + 1, rows1_v, gsem1)
        scat(jj, rows0_v)
        wait_g(rows1_v, gsem1)
        start_g(jj + 2, rows0_v, gsem0)
        scat(jj + 1, rows1_v)

      wait_g(rows0_v, gsem0)
      start_g(sg - 1, rows1_v, gsem1)
      scat(sg - 2, rows0_v)
      wait_g(rows1_v, gsem1)
      scat(sg - 1, rows1_v)

    plsc.subcore_barrier()

    pltpu.sync_copy(accum_sh.at[pl.ds(s * rps, rps)],
                    out_hbm.at[c].at[pl.ds(s * rps, rps)])

  return agg_kernel


def _dinv_of(degp_block):
  deg = jnp.sum(degp_block, axis=0) + 1.0
  return lax.rsqrt(deg)


def _tc1_body(degp_ref, x_ref, w_ref, y_ref):
  dinv = _dinv_of(degp_ref[...])
  xw = jnp.dot(x_ref[...], w_ref[...], preferred_element_type=jnp.float32)
  y_ref[...] = xw * dinv[:, None]


def _tc2_body(degp_ref, p_ref, y_ref, b_ref, w_ref, o_ref):
  dinv = _dinv_of(degp_ref[...])
  pre = (p_ref[0] + p_ref[1] + y_ref[...]) * dinv[:, None] + b_ref[...]
  h = jnp.maximum(pre, 0.0)
  o_ref[...] = jnp.dot(h, w_ref[...], preferred_element_type=jnp.float32) \
      * dinv[:, None]


def _tc3_body(degp_ref, p_ref, y_ref, b_ref, wl_ref, bl_ref, o_ref):
  dinv = _dinv_of(degp_ref[...])
  pre = (p_ref[0] + p_ref[1] + y_ref[...]) * dinv[:, None] + b_ref[...]
  h = jnp.maximum(pre, 0.0)
  o_ref[...] = jnp.dot(h, wl_ref[...], preferred_element_type=jnp.float32) \
      + bl_ref[...]


def kernel(x, edge_index, W1, b1, W2, b2, Wl, bl):
  n, d = x.shape
  h = W1.shape[1]
  e = edge_index.shape[1]

  cpt = 2 * (-(-e // (NW * K * 2)))
  e_pad = NW * cpt * K
  n_pad = ((n + 1 + NS * ZR - 1) // (NS * ZR)) * (NS * ZR)

  src = jnp.concatenate(
      [edge_index[0], jnp.zeros((e_pad - e,), jnp.int32)]).reshape(NW, cpt, K)
  dst = jnp.concatenate(
      [edge_index[1], jnp.full((e_pad - e,), n, jnp.int32)]).reshape(NW, cpt, K)

  xp = jnp.pad(x, ((0, n_pad - n), (0, 0)))
  b1r = b1.reshape(1, h)
  b2r = b2.reshape(1, h)
  blr = bl.reshape(1, 1)

  deg_kernel = _make_deg_kernel(n_pad, cpt)
  agg_kernel = _make_agg_kernel(n_pad, cpt, h)

  degp = deg_kernel(dst)

  bn = 1024
  grid = (n_pad // bn,)
  degp_spec = pl.BlockSpec((NW, bn), lambda i: (0, i))
  row_spec = pl.BlockSpec((bn, d), lambda i: (i, 0))
  p_spec = pl.BlockSpec((NC, bn, h), lambda i: (0, i, 0))
  w_spec = pl.BlockSpec((d, h), lambda i: (0, 0))
  b_spec = pl.BlockSpec((1, h), lambda i: (0, 0))

  y1 = pl.pallas_call(
      _tc1_body,
      grid=grid,
      in_specs=[degp_spec, row_spec, w_spec],
      out_specs=pl.BlockSpec((bn, h), lambda i: (i, 0)),
      out_shape=jax.ShapeDtypeStruct((n_pad, h), jnp.float32),
  )(degp, xp, W1)

  p1 = agg_kernel(y1, src, dst)

  y2 = pl.pallas_call(
      _tc2_body,
      grid=grid,
      in_specs=[degp_spec, p_spec, row_spec, b_spec, w_spec],
      out_specs=pl.BlockSpec((bn, h), lambda i: (i, 0)),
      out_shape=jax.ShapeDtypeStruct((n_pad, h), jnp.float32),
  )(degp, p1, y1, b1r, W2)

  p2 = agg_kernel(y2, src, dst)

  out = pl.pallas_call(
      _tc3_body,
      grid=grid,
      in_specs=[degp_spec, p_spec, row_spec, b_spec,
                pl.BlockSpec((h, 1), lambda i: (0, 0)),
                pl.BlockSpec((1, 1), lambda i: (0, 0))],
      out_specs=pl.BlockSpec((bn, 1), lambda i: (i, 0)),
      out_shape=jax.ShapeDtypeStruct((n_pad, 1), jnp.float32),
  )(degp, p2, y2, b2r, Wl, blr)

  return out[:n]


# speedup vs baseline: 1.0237x; 1.0237x over previous
import dataclasses
import functools

import jax
import jax.numpy as jnp
from jax import lax
from jax.experimental import pallas as pl
from jax.experimental.pallas import tpu as pltpu
from jax.experimental.pallas import tpu_sc as plsc

NC = 2
NS = 16
NW = NC * NS
LANES = 16
K = 128
ZR = 16
SG = 8
F0 = 0.65


def _splits(e):
  ctot = -(-e // (NS * K))
  ctot = ((ctot + 2 * SG - 1) // (2 * SG)) * (2 * SG)
  cpt0 = int(round(F0 * ctot / SG)) * SG
  cpt0 = min(max(cpt0, SG), ctot - SG)
  return cpt0, ctot - cpt0


def _sc_mesh():
  return plsc.VectorSubcoreMesh(core_axis_name="c", subcore_axis_name="s")


def _sc_params():
  cp = pltpu.CompilerParams()
  if "needs_layout_passes" in pltpu.CompilerParams.__dataclass_fields__:
    cp = dataclasses.replace(cp, needs_layout_passes=False)
  return cp


def _chunk_range(c, s, cpt0, cpt1):
  base = jnp.where(c == 0, s * cpt0, NS * cpt0 + s * cpt1)
  nstage = jnp.where(c == 0, cpt0 // SG, cpt1 // SG)
  return base, nstage


def _make_deg_kernel(n_pad, cpt0, cpt1):
  @functools.partial(
      pl.kernel,
      out_type=jax.ShapeDtypeStruct((NW, n_pad), jnp.float32),
      mesh=_sc_mesh(),
      scratch_types=[
          pltpu.VMEM((n_pad,), jnp.float32),
          pltpu.VMEM((SG, K), jnp.int32),
      ],
      compiler_params=_sc_params(),
  )
  def deg_kernel(dst_hbm, out_hbm, deg_v, idx_v):
    c = lax.axis_index("c")
    s = lax.axis_index("s")
    wid = c * NS + s
    base, nstage = _chunk_range(c, s, cpt0, cpt1)
    zeros16 = jnp.zeros((LANES,), jnp.float32)
    ones16 = jnp.ones((LANES,), jnp.float32)

    @pl.loop(0, n_pad // LANES)
    def _(i):
      deg_v[pl.ds(i * LANES, LANES)] = zeros16

    @pl.loop(0, nstage)
    def _(st):
      pltpu.sync_copy(dst_hbm.at[pl.ds(base + st * SG, SG)], idx_v)

      @pl.loop(0, SG)
      def _(j):
        for q in range(K // LANES):
          idx = idx_v[j, pl.ds(q * LANES, LANES)]
          plsc.addupdate_scatter(deg_v, [idx], ones16)

    pltpu.sync_copy(deg_v, out_hbm.at[wid])

  return deg_kernel


def _make_agg_kernel(n_pad, cpt0, cpt1, dh):
  rps = n_pad // NS

  @functools.partial(
      pl.kernel,
      out_type=jax.ShapeDtypeStruct((NC, n_pad, dh), jnp.float32),
      mesh=_sc_mesh(),
      scratch_types=[
          pltpu.VMEM((SG, K), jnp.int32),
          pltpu.VMEM((SG, K), jnp.int32),
          pltpu.VMEM((K, dh), jnp.float32),
          pltpu.VMEM((ZR, dh), jnp.float32),
          pltpu.VMEM_SHARED((n_pad, dh), jnp.float32),
          pltpu.SemaphoreType.DMA,
      ],
      compiler_params=_sc_params(),
  )
  def agg_kernel(y_hbm, src_hbm, dst_hbm, out_hbm,
                 src_v, dst_v, rows_v, zero_v, accum_sh, gsem):
    c = lax.axis_index("c")
    s = lax.axis_index("s")
    base, nstage = _chunk_range(c, s, cpt0, cpt1)
    zeros16 = jnp.zeros((LANES,), jnp.float32)

    @pl.loop(0, ZR)
    def _(r):
      for q in range(dh // LANES):
        zero_v[r, pl.ds(q * LANES, LANES)] = zeros16

    @pl.loop(0, rps // ZR)
    def _(z):
      pltpu.sync_copy(zero_v, accum_sh.at[pl.ds(s * rps + z * ZR, ZR)])

    plsc.subcore_barrier()

    @pl.loop(0, nstage)
    def _(st):
      pltpu.sync_copy(src_hbm.at[pl.ds(base + st * SG, SG)], src_v)
      pltpu.sync_copy(dst_hbm.at[pl.ds(base + st * SG, SG)], dst_v)

      @pl.loop(0, SG)
      def _(j):
        pltpu.async_copy(y_hbm.at[src_v.at[j]], rows_v, gsem).wait()
        pltpu.sync_copy(rows_v, accum_sh.at[dst_v.at[j]], add=True)

    plsc.subcore_barrier()

    pltpu.sync_copy(accum_sh.at[pl.ds(s * rps, rps)],
                    out_hbm.at[c].at[pl.ds(s * rps, rps)])

  return agg_kernel


def _dinv_of(degp_block):
  deg = jnp.sum(degp_block, axis=0) + 1.0
  return lax.rsqrt(deg)


def _tc1_body(degp_ref, x_ref, w_ref, y_ref):
  dinv = _dinv_of(degp_ref[...])
  xw = jnp.dot(x_ref[...], w_ref[...], preferred_element_type=jnp.float32)
  y_ref[...] = xw * dinv[:, None]


def _tc2_body(degp_ref, p_ref, y_ref, b_ref, w_ref, o_ref):
  dinv = _dinv_of(degp_ref[...])
  pre = (p_ref[0] + p_ref[1] + y_ref[...]) * dinv[:, None] + b_ref[...]
  h = jnp.maximum(pre, 0.0)
  o_ref[...] = jnp.dot(h, w_ref[...], preferred_element_type=jnp.float32) \
      * dinv[:, None]


def _tc3_body(degp_ref, p_ref, y_ref, b_ref, wl_ref, bl_ref, o_ref):
  dinv = _dinv_of(degp_ref[...])
  pre = (p_ref[0] + p_ref[1] + y_ref[...]) * dinv[:, None] + b_ref[...]
  h = jnp.maximum(pre, 0.0)
  o_ref[...] = jnp.dot(h, wl_ref[...], preferred_element_type=jnp.float32) \
      + bl_ref[...]


def kernel(x, edge_index, W1, b1, W2, b2, Wl, bl):
  n, d = x.shape
  h = W1.shape[1]
  e = edge_index.shape[1]

  cpt0, cpt1 = _splits(e)
  nchunks = NS * (cpt0 + cpt1)
  e_pad = nchunks * K
  n_pad = ((n + 1 + NS * ZR - 1) // (NS * ZR)) * (NS * ZR)

  src = jnp.concatenate(
      [edge_index[0], jnp.zeros((e_pad - e,), jnp.int32)]).reshape(nchunks, K)
  dst = jnp.concatenate(
      [edge_index[1], jnp.full((e_pad - e,), n, jnp.int32)]).reshape(nchunks, K)

  xp = jnp.pad(x, ((0, n_pad - n), (0, 0)))
  b1r = b1.reshape(1, h)
  b2r = b2.reshape(1, h)
  blr = bl.reshape(1, 1)

  deg_kernel = _make_deg_kernel(n_pad, cpt0, cpt1)
  agg_kernel = _make_agg_kernel(n_pad, cpt0, cpt1, h)

  degp = deg_kernel(dst)

  bn = 1024
  grid = (n_pad // bn,)
  degp_spec = pl.BlockSpec((NW, bn), lambda i: (0, i))
  row_spec = pl.BlockSpec((bn, d), lambda i: (i, 0))
  p_spec = pl.BlockSpec((NC, bn, h), lambda i: (0, i, 0))
  w_spec = pl.BlockSpec((d, h), lambda i: (0, 0))
  b_spec = pl.BlockSpec((1, h), lambda i: (0, 0))

  y1 = pl.pallas_call(
      _tc1_body,
      grid=grid,
      in_specs=[degp_spec, row_spec, w_spec],
      out_specs=pl.BlockSpec((bn, h), lambda i: (i, 0)),
      out_shape=jax.ShapeDtypeStruct((n_pad, h), jnp.float32),
  )(degp, xp, W1)

  p1 = agg_kernel(y1, src, dst)

  y2 = pl.pallas_call(
      _tc2_body,
      grid=grid,
      in_specs=[degp_spec, p_spec, row_spec, b_spec, w_spec],
      out_specs=pl.BlockSpec((bn, h), lambda i: (i, 0)),
      out_shape=jax.ShapeDtypeStruct((n_pad, h), jnp.float32),
  )(degp, p1, y1, b1r, W2)

  p2 = agg_kernel(y2, src, dst)

  out = pl.pallas_call(
      _tc3_body,
      grid=grid,
      in_specs=[degp_spec, p_spec, row_spec, b_spec,
                pl.BlockSpec((h, 1), lambda i: (0, 0)),
                pl.BlockSpec((1, 1), lambda i: (0, 0))],
      out_specs=pl.BlockSpec((bn, 1), lambda i: (i, 0)),
      out_shape=jax.ShapeDtypeStruct((n_pad, 1), jnp.float32),
  )(degp, p2, y2, b2r, Wl, blr)

  return out[:n]


# speedup vs baseline: 1.0346x; 1.0106x over previous
import dataclasses
import functools

import jax
import jax.numpy as jnp
from jax import lax
from jax.experimental import pallas as pl
from jax.experimental.pallas import tpu as pltpu
from jax.experimental.pallas import tpu_sc as plsc

NC = 2
NS = 16
NW = NC * NS
LANES = 16
K = 128
ZR = 16
SG = 8
F0 = 0.65


def _splits(e):
  ctot = -(-e // (NS * K))
  ctot = ((ctot + 2 * SG - 1) // (2 * SG)) * (2 * SG)
  cpt0 = int(round(F0 * ctot / SG)) * SG
  cpt0 = min(max(cpt0, SG), ctot - SG)
  return cpt0, ctot - cpt0


def _sc_mesh():
  return plsc.VectorSubcoreMesh(core_axis_name="c", subcore_axis_name="s")


def _sc_params():
  cp = pltpu.CompilerParams()
  if "needs_layout_passes" in pltpu.CompilerParams.__dataclass_fields__:
    cp = dataclasses.replace(cp, needs_layout_passes=False)
  return cp


def _chunk_range(c, s, cpt0, cpt1):
  base = jnp.where(c == 0, s * cpt0, NS * cpt0 + s * cpt1)
  my_cpt = jnp.where(c == 0, cpt0, cpt1)
  return base, my_cpt


def _make_deg_kernel(n_pad, cpt0, cpt1):
  cptm = max(cpt0, cpt1)

  @functools.partial(
      pl.kernel,
      out_type=jax.ShapeDtypeStruct((NW, n_pad), jnp.float32),
      mesh=_sc_mesh(),
      scratch_types=[
          pltpu.VMEM((n_pad,), jnp.float32),
          pltpu.VMEM((cptm, K), jnp.int32),
      ],
      compiler_params=_sc_params(),
  )
  def deg_kernel(dst_hbm, out_hbm, deg_v, idx_v):
    c = lax.axis_index("c")
    s = lax.axis_index("s")
    wid = c * NS + s
    base, my_cpt = _chunk_range(c, s, cpt0, cpt1)
    zeros16 = jnp.zeros((LANES,), jnp.float32)
    ones16 = jnp.ones((LANES,), jnp.float32)

    @pl.loop(0, n_pad // LANES)
    def _(i):
      deg_v[pl.ds(i * LANES, LANES)] = zeros16

    @pl.when(c == 0)
    def _():
      pltpu.sync_copy(dst_hbm.at[pl.ds(base, cpt0)], idx_v.at[pl.ds(0, cpt0)])

    @pl.when(c == 1)
    def _():
      pltpu.sync_copy(dst_hbm.at[pl.ds(base, cpt1)], idx_v.at[pl.ds(0, cpt1)])

    @pl.loop(0, my_cpt)
    def _(j):
      for q in range(K // LANES):
        idx = idx_v[j, pl.ds(q * LANES, LANES)]
        plsc.addupdate_scatter(deg_v, [idx], ones16)

    pltpu.sync_copy(deg_v, out_hbm.at[wid])

  return deg_kernel


def _make_agg_kernel(n_pad, cpt0, cpt1, dh):
  rps = n_pad // NS
  cptm = max(cpt0, cpt1)

  @functools.partial(
      pl.kernel,
      out_type=jax.ShapeDtypeStruct((NC, n_pad, dh), jnp.float32),
      mesh=_sc_mesh(),
      scratch_types=[
          pltpu.VMEM((cptm, K), jnp.int32),
          pltpu.VMEM((cptm, K), jnp.int32),
          pltpu.VMEM((K, dh), jnp.float32),
          pltpu.VMEM((ZR, dh), jnp.float32),
          pltpu.VMEM_SHARED((n_pad, dh), jnp.float32),
          pltpu.SemaphoreType.DMA,
      ],
      compiler_params=_sc_params(),
  )
  def agg_kernel(y_hbm, src_hbm, dst_hbm, out_hbm,
                 src_v, dst_v, rows_v, zero_v, accum_sh, gsem):
    c = lax.axis_index("c")
    s = lax.axis_index("s")
    base, my_cpt = _chunk_range(c, s, cpt0, cpt1)
    zeros16 = jnp.zeros((LANES,), jnp.float32)

    @pl.loop(0, ZR)
    def _(r):
      for q in range(dh // LANES):
        zero_v[r, pl.ds(q * LANES, LANES)] = zeros16

    @pl.loop(0, rps // ZR)
    def _(z):
      pltpu.sync_copy(zero_v, accum_sh.at[pl.ds(s * rps + z * ZR, ZR)])

    @pl.when(c == 0)
    def _():
      pltpu.sync_copy(src_hbm.at[pl.ds(base, cpt0)], src_v.at[pl.ds(0, cpt0)])
      pltpu.sync_copy(dst_hbm.at[pl.ds(base, cpt0)], dst_v.at[pl.ds(0, cpt0)])

    @pl.when(c == 1)
    def _():
      pltpu.sync_copy(src_hbm.at[pl.ds(base, cpt1)], src_v.at[pl.ds(0, cpt1)])
      pltpu.sync_copy(dst_hbm.at[pl.ds(base, cpt1)], dst_v.at[pl.ds(0, cpt1)])

    plsc.subcore_barrier()

    @pl.loop(0, my_cpt)
    def _(j):
      pltpu.async_copy(y_hbm.at[src_v.at[j]], rows_v, gsem).wait()
      pltpu.sync_copy(rows_v, accum_sh.at[dst_v.at[j]], add=True)

    plsc.subcore_barrier()

    pltpu.sync_copy(accum_sh.at[pl.ds(s * rps, rps)],
                    out_hbm.at[c].at[pl.ds(s * rps, rps)])

  return agg_kernel


def _dinv_of(degp_block):
  deg = jnp.sum(degp_block, axis=0) + 1.0
  return lax.rsqrt(deg)


def _tc1_body(degp_ref, x_ref, w_ref, y_ref):
  dinv = _dinv_of(degp_ref[...])
  xw = jnp.dot(x_ref[...], w_ref[...], preferred_element_type=jnp.float32)
  y_ref[...] = xw * dinv[:, None]


def _tc2_body(degp_ref, p_ref, y_ref, b_ref, w_ref, o_ref):
  dinv = _dinv_of(degp_ref[...])
  pre = (p_ref[0] + p_ref[1] + y_ref[...]) * dinv[:, None] + b_ref[...]
  h = jnp.maximum(pre, 0.0)
  o_ref[...] = jnp.dot(h, w_ref[...], preferred_element_type=jnp.float32) \
      * dinv[:, None]


def _tc3_body(degp_ref, p_ref, y_ref, b_ref, wl_ref, bl_ref, o_ref):
  dinv = _dinv_of(degp_ref[...])
  pre = (p_ref[0] + p_ref[1] + y_ref[...]) * dinv[:, None] + b_ref[...]
  h = jnp.maximum(pre, 0.0)
  o_ref[...] = jnp.dot(h, wl_ref[...], preferred_element_type=jnp.float32) \
      + bl_ref[...]


def kernel(x, edge_index, W1, b1, W2, b2, Wl, bl):
  n, d = x.shape
  h = W1.shape[1]
  e = edge_index.shape[1]

  cpt0, cpt1 = _splits(e)
  nchunks = NS * (cpt0 + cpt1)
  e_pad = nchunks * K
  n_pad = ((n + 1 + NS * ZR - 1) // (NS * ZR)) * (NS * ZR)

  src = jnp.concatenate(
      [edge_index[0], jnp.zeros((e_pad - e,), jnp.int32)]).reshape(nchunks, K)
  dst = jnp.concatenate(
      [edge_index[1], jnp.full((e_pad - e,), n, jnp.int32)]).reshape(nchunks, K)

  xp = jnp.pad(x, ((0, n_pad - n), (0, 0)))
  b1r = b1.reshape(1, h)
  b2r = b2.reshape(1, h)
  blr = bl.reshape(1, 1)

  deg_kernel = _make_deg_kernel(n_pad, cpt0, cpt1)
  agg_kernel = _make_agg_kernel(n_pad, cpt0, cpt1, h)

  degp = deg_kernel(dst)

  bn = 1024
  grid = (n_pad // bn,)
  degp_spec = pl.BlockSpec((NW, bn), lambda i: (0, i))
  row_spec = pl.BlockSpec((bn, d), lambda i: (i, 0))
  p_spec = pl.BlockSpec((NC, bn, h), lambda i: (0, i, 0))
  w_spec = pl.BlockSpec((d, h), lambda i: (0, 0))
  b_spec = pl.BlockSpec((1, h), lambda i: (0, 0))

  y1 = pl.pallas_call(
      _tc1_body,
      grid=grid,
      in_specs=[degp_spec, row_spec, w_spec],
      out_specs=pl.BlockSpec((bn, h), lambda i: (i, 0)),
      out_shape=jax.ShapeDtypeStruct((n_pad, h), jnp.float32),
  )(degp, xp, W1)

  p1 = agg_kernel(y1, src, dst)

  y2 = pl.pallas_call(
      _tc2_body,
      grid=grid,
      in_specs=[degp_spec, p_spec, row_spec, b_spec, w_spec],
      out_specs=pl.BlockSpec((bn, h), lambda i: (i, 0)),
      out_shape=jax.ShapeDtypeStruct((n_pad, h), jnp.float32),
  )(degp, p1, y1, b1r, W2)

  p2 = agg_kernel(y2, src, dst)

  out = pl.pallas_call(
      _tc3_body,
      grid=grid,
      in_specs=[degp_spec, p_spec, row_spec, b_spec,
                pl.BlockSpec((h, 1), lambda i: (0, 0)),
                pl.BlockSpec((1, 1), lambda i: (0, 0))],
      out_specs=pl.BlockSpec((bn, 1), lambda i: (i, 0)),
      out_shape=jax.ShapeDtypeStruct((n_pad, 1), jnp.float32),
  )(degp, p2, y2, b2r, Wl, blr)

  return out[:n]


# speedup vs baseline: 1.0363x; 1.0017x over previous
import dataclasses
import functools

import jax
import jax.numpy as jnp
from jax import lax
from jax.experimental import pallas as pl
from jax.experimental.pallas import tpu as pltpu
from jax.experimental.pallas import tpu_sc as plsc

NC = 2
NS = 16
NW = NC * NS
LANES = 16
K = 128
ZR = 16
SG = 8
F0 = 0.65


def _splits(e):
  ctot = -(-e // (NS * K))
  ctot = ((ctot + 2 * SG - 1) // (2 * SG)) * (2 * SG)
  cpt0 = int(round(F0 * ctot / SG)) * SG
  cpt0 = min(max(cpt0, SG), ctot - SG)
  return cpt0, ctot - cpt0


def _sc_mesh():
  return plsc.VectorSubcoreMesh(core_axis_name="c", subcore_axis_name="s")


def _sc_params():
  cp = pltpu.CompilerParams()
  if "needs_layout_passes" in pltpu.CompilerParams.__dataclass_fields__:
    cp = dataclasses.replace(cp, needs_layout_passes=False)
  return cp


def _core_base(c_static, s, cpt0, cpt1):
  if c_static == 0:
    return s * cpt0
  return NS * cpt0 + s * cpt1


def _make_deg_kernel(n_pad, cpt0, cpt1):
  cptm = max(cpt0, cpt1)

  @functools.partial(
      pl.kernel,
      out_type=jax.ShapeDtypeStruct((NW, n_pad), jnp.float32),
      mesh=_sc_mesh(),
      scratch_types=[
          pltpu.VMEM((n_pad,), jnp.float32),
          pltpu.VMEM((cptm, K), jnp.int32),
      ],
      compiler_params=_sc_params(),
  )
  def deg_kernel(dst_hbm, out_hbm, deg_v, idx_v):
    c = lax.axis_index("c")
    s = lax.axis_index("s")
    wid = c * NS + s
    zeros16 = jnp.zeros((LANES,), jnp.float32)
    ones16 = jnp.ones((LANES,), jnp.float32)

    @pl.loop(0, n_pad // LANES)
    def _(i):
      deg_v[pl.ds(i * LANES, LANES)] = zeros16

    def core_body(cc, cpt):
      base = _core_base(cc, s, cpt0, cpt1)
      pltpu.sync_copy(dst_hbm.at[pl.ds(base, cpt)], idx_v.at[pl.ds(0, cpt)])

      @pl.loop(0, cpt)
      def _(j):
        for q in range(K // LANES):
          idx = idx_v[j, pl.ds(q * LANES, LANES)]
          plsc.addupdate_scatter(deg_v, [idx], ones16)

    @pl.when(c == 0)
    def _():
      core_body(0, cpt0)

    @pl.when(c == 1)
    def _():
      core_body(1, cpt1)

    pltpu.sync_copy(deg_v, out_hbm.at[wid])

  return deg_kernel


def _make_agg_kernel(n_pad, cpt0, cpt1, dh):
  rps = n_pad // NS
  cptm = max(cpt0, cpt1)

  @functools.partial(
      pl.kernel,
      out_type=jax.ShapeDtypeStruct((NC, n_pad, dh), jnp.float32),
      mesh=_sc_mesh(),
      scratch_types=[
          pltpu.VMEM((cptm, K), jnp.int32),
          pltpu.VMEM((cptm, K), jnp.int32),
          pltpu.VMEM((K, dh), jnp.float32),
          pltpu.VMEM((ZR, dh), jnp.float32),
          pltpu.VMEM_SHARED((n_pad, dh), jnp.float32),
          pltpu.SemaphoreType.DMA,
      ],
      compiler_params=_sc_params(),
  )
  def agg_kernel(y_hbm, src_hbm, dst_hbm, out_hbm,
                 src_v, dst_v, rows_v, zero_v, accum_sh, gsem):
    c = lax.axis_index("c")
    s = lax.axis_index("s")
    zeros16 = jnp.zeros((LANES,), jnp.float32)

    @pl.loop(0, ZR)
    def _(r):
      for q in range(dh // LANES):
        zero_v[r, pl.ds(q * LANES, LANES)] = zeros16

    @pl.loop(0, rps // ZR)
    def _(z):
      pltpu.sync_copy(zero_v, accum_sh.at[pl.ds(s * rps + z * ZR, ZR)])

    plsc.subcore_barrier()

    def core_body(cc, cpt):
      base = _core_base(cc, s, cpt0, cpt1)
      pltpu.sync_copy(src_hbm.at[pl.ds(base, cpt)], src_v.at[pl.ds(0, cpt)])
      pltpu.sync_copy(dst_hbm.at[pl.ds(base, cpt)], dst_v.at[pl.ds(0, cpt)])

      @pl.loop(0, cpt)
      def _(j):
        pltpu.async_copy(y_hbm.at[src_v.at[j]], rows_v, gsem).wait()
        pltpu.sync_copy(rows_v, accum_sh.at[dst_v.at[j]], add=True)

    @pl.when(c == 0)
    def _():
      core_body(0, cpt0)

    @pl.when(c == 1)
    def _():
      core_body(1, cpt1)

    plsc.subcore_barrier()

    pltpu.sync_copy(accum_sh.at[pl.ds(s * rps, rps)],
                    out_hbm.at[c].at[pl.ds(s * rps, rps)])

  return agg_kernel


def _dinv_of(degp_block):
  deg = jnp.sum(degp_block, axis=0) + 1.0
  return lax.rsqrt(deg)


def _tc1_body(degp_ref, x_ref, w_ref, y_ref):
  dinv = _dinv_of(degp_ref[...])
  xw = jnp.dot(x_ref[...], w_ref[...], preferred_element_type=jnp.float32)
  y_ref[...] = xw * dinv[:, None]


def _tc2_body(degp_ref, p_ref, y_ref, b_ref, w_ref, o_ref):
  dinv = _dinv_of(degp_ref[...])
  pre = (p_ref[0] + p_ref[1] + y_ref[...]) * dinv[:, None] + b_ref[...]
  h = jnp.maximum(pre, 0.0)
  o_ref[...] = jnp.dot(h, w_ref[...], preferred_element_type=jnp.float32) \
      * dinv[:, None]


def _tc3_body(degp_ref, p_ref, y_ref, b_ref, wl_ref, bl_ref, o_ref):
  dinv = _dinv_of(degp_ref[...])
  pre = (p_ref[0] + p_ref[1] + y_ref[...]) * dinv[:, None] + b_ref[...]
  h = jnp.maximum(pre, 0.0)
  o_ref[...] = jnp.dot(h, wl_ref[...], preferred_element_type=jnp.float32) \
      + bl_ref[...]


def kernel(x, edge_index, W1, b1, W2, b2, Wl, bl):
  n, d = x.shape
  h = W1.shape[1]
  e = edge_index.shape[1]

  cpt0, cpt1 = _splits(e)
  nchunks = NS * (cpt0 + cpt1)
  e_pad = nchunks * K
  n_pad = ((n + 1 + NS * ZR - 1) // (NS * ZR)) * (NS * ZR)

  src = jnp.concatenate(
      [edge_index[0], jnp.zeros((e_pad - e,), jnp.int32)]).reshape(nchunks, K)
  dst = jnp.concatenate(
      [edge_index[1], jnp.full((e_pad - e,), n, jnp.int32)]).reshape(nchunks, K)

  xp = jnp.pad(x, ((0, n_pad - n), (0, 0)))
  b1r = b1.reshape(1, h)
  b2r = b2.reshape(1, h)
  blr = bl.reshape(1, 1)

  deg_kernel = _make_deg_kernel(n_pad, cpt0, cpt1)
  agg_kernel = _make_agg_kernel(n_pad, cpt0, cpt1, h)

  degp = deg_kernel(dst)

  bn = 1024
  grid = (n_pad // bn,)
  degp_spec = pl.BlockSpec((NW, bn), lambda i: (0, i))
  row_spec = pl.BlockSpec((bn, d), lambda i: (i, 0))
  p_spec = pl.BlockSpec((NC, bn, h), lambda i: (0, i, 0))
  w_spec = pl.BlockSpec((d, h), lambda i: (0, 0))
  b_spec = pl.BlockSpec((1, h), lambda i: (0, 0))

  y1 = pl.pallas_call(
      _tc1_body,
      grid=grid,
      in_specs=[degp_spec, row_spec, w_spec],
      out_specs=pl.BlockSpec((bn, h), lambda i: (i, 0)),
      out_shape=jax.ShapeDtypeStruct((n_pad, h), jnp.float32),
  )(degp, xp, W1)

  p1 = agg_kernel(y1, src, dst)

  y2 = pl.pallas_call(
      _tc2_body,
      grid=grid,
      in_specs=[degp_spec, p_spec, row_spec, b_spec, w_spec],
      out_specs=pl.BlockSpec((bn, h), lambda i: (i, 0)),
      out_shape=jax.ShapeDtypeStruct((n_pad, h), jnp.float32),
  )(degp, p1, y1, b1r, W2)

  p2 = agg_kernel(y2, src, dst)

  out = pl.pallas_call(
      _tc3_body,
      grid=grid,
      in_specs=[degp_spec, p_spec, row_spec, b_spec,
                pl.BlockSpec((h, 1), lambda i: (0, 0)),
                pl.BlockSpec((1, 1), lambda i: (0, 0))],
      out_specs=pl.BlockSpec((bn, 1), lambda i: (i, 0)),
      out_shape=jax.ShapeDtypeStruct((n_pad, 1), jnp.float32),
  )(degp, p2, y2, b2r, Wl, blr)

  return out[:n]


# speedup vs baseline: 2.0422x; 1.9707x over previous
import dataclasses
import functools

import jax
import jax.numpy as jnp
from jax import lax
from jax.experimental import pallas as pl
from jax.experimental.pallas import tpu as pltpu
from jax.experimental.pallas import tpu_sc as plsc

NC = 2
NS = 16
NW = NC * NS
LANES = 16
K = 128
ZR = 16
SG = 8
F0 = 0.65


def _splits(e):
  ctot = -(-e // (NS * K))
  ctot = ((ctot + 2 * SG - 1) // (2 * SG)) * (2 * SG)
  cpt0 = int(round(F0 * ctot / SG)) * SG
  cpt0 = min(max(cpt0, SG), ctot - SG)
  return cpt0, ctot - cpt0


def _sc_mesh():
  return plsc.VectorSubcoreMesh(core_axis_name="c", subcore_axis_name="s")


def _sc_params():
  cp = pltpu.CompilerParams()
  if "needs_layout_passes" in pltpu.CompilerParams.__dataclass_fields__:
    cp = dataclasses.replace(cp, needs_layout_passes=False)
  return cp


def _core_base(c_static, s, cpt0, cpt1):
  if c_static == 0:
    return s * cpt0
  return NS * cpt0 + s * cpt1


def _make_deg_kernel(n_pad, cpt0, cpt1):
  cptm = max(cpt0, cpt1)

  @functools.partial(
      pl.kernel,
      out_type=jax.ShapeDtypeStruct((NW, n_pad), jnp.float32),
      mesh=_sc_mesh(),
      scratch_types=[
          pltpu.VMEM((n_pad,), jnp.float32),
          pltpu.VMEM((cptm, K), jnp.int32),
      ],
      compiler_params=_sc_params(),
  )
  def deg_kernel(dst_hbm, out_hbm, deg_v, idx_v):
    c = lax.axis_index("c")
    s = lax.axis_index("s")
    wid = c * NS + s
    zeros16 = jnp.zeros((LANES,), jnp.float32)
    ones16 = jnp.ones((LANES,), jnp.float32)

    @pl.loop(0, n_pad // LANES)
    def _(i):
      deg_v[pl.ds(i * LANES, LANES)] = zeros16

    def core_body(cc, cpt):
      base = _core_base(cc, s, cpt0, cpt1)
      pltpu.sync_copy(dst_hbm.at[pl.ds(base, cpt)], idx_v.at[pl.ds(0, cpt)])

      @pl.loop(0, cpt)
      def _(j):
        for q in range(K // LANES):
          idx = idx_v[j, pl.ds(q * LANES, LANES)]
          plsc.addupdate_scatter(deg_v, [idx], ones16)

    @pl.when(c == 0)
    def _():
      core_body(0, cpt0)

    @pl.when(c == 1)
    def _():
      core_body(1, cpt1)

    pltpu.sync_copy(deg_v, out_hbm.at[wid])

  return deg_kernel


def _make_agg_kernel(n_pad, cpt0, cpt1, dh):
  rps = n_pad // NS
  cptm = max(cpt0, cpt1)

  @functools.partial(
      pl.kernel,
      out_type=jax.ShapeDtypeStruct((NC, n_pad, dh), jnp.float32),
      mesh=_sc_mesh(),
      scratch_types=[
          pltpu.VMEM((cptm, K), jnp.int32),
          pltpu.VMEM((cptm, K), jnp.int32),
          pltpu.VMEM((K, dh), jnp.float32),
          pltpu.VMEM((ZR, dh), jnp.float32),
          pltpu.VMEM_SHARED((n_pad, dh), jnp.float32),
          pltpu.SemaphoreType.DMA,
      ],
      compiler_params=_sc_params(),
  )
  def agg_kernel(y_hbm, src_hbm, dst_hbm, out_hbm,
                 src_v, dst_v, rows_v, zero_v, accum_sh, gsem):
    c = lax.axis_index("c")
    s = lax.axis_index("s")
    zeros16 = jnp.zeros((LANES,), jnp.float32)

    @pl.loop(0, ZR)
    def _(r):
      for q in range(dh // LANES):
        zero_v[r, pl.ds(q * LANES, LANES)] = zeros16

    @pl.loop(0, rps // ZR)
    def _(z):
      pltpu.sync_copy(zero_v, accum_sh.at[pl.ds(s * rps + z * ZR, ZR)])

    plsc.subcore_barrier()

    def core_body(cc, cpt):
      base = _core_base(cc, s, cpt0, cpt1)
      pltpu.sync_copy(src_hbm.at[pl.ds(base, cpt)], src_v.at[pl.ds(0, cpt)])
      pltpu.sync_copy(dst_hbm.at[pl.ds(base, cpt)], dst_v.at[pl.ds(0, cpt)])

      @pl.loop(0, cpt)
      def _(j):
        pltpu.async_copy(y_hbm.at[src_v.at[j]], rows_v, gsem).wait()
        pltpu.sync_copy(rows_v, accum_sh.at[dst_v.at[j]], add=True)

    @pl.when(c == 0)
    def _():
      core_body(0, cpt0)

    @pl.when(c == 1)
    def _():
      core_body(1, cpt1)

    plsc.subcore_barrier()

    pltpu.sync_copy(accum_sh.at[pl.ds(s * rps, rps)],
                    out_hbm.at[c].at[pl.ds(s * rps, rps)])

  return agg_kernel


def _dinv_of(degp_block):
  deg = jnp.sum(degp_block, axis=0) + 1.0
  return lax.rsqrt(deg)


def _tc1_body(degp_ref, x_ref, w_ref, y_ref):
  dinv = _dinv_of(degp_ref[...])
  xw = jnp.dot(x_ref[...], w_ref[...], preferred_element_type=jnp.float32)
  y_ref[...] = xw * dinv[:, None]


def _tc2_body(degp_ref, p_ref, y_ref, b_ref, w_ref, o_ref):
  dinv = _dinv_of(degp_ref[...])
  pre = (p_ref[0] + p_ref[1] + y_ref[...]) * dinv[:, None] + b_ref[...]
  h = jnp.maximum(pre, 0.0)
  o_ref[...] = jnp.dot(h, w_ref[...], preferred_element_type=jnp.float32) \
      * dinv[:, None]


def _tc3_body(degp_ref, p_ref, y_ref, b_ref, wl_ref, bl_ref, o_ref):
  dinv = _dinv_of(degp_ref[...])
  pre = (p_ref[0] + p_ref[1] + y_ref[...]) * dinv[:, None] + b_ref[...]
  h = jnp.maximum(pre, 0.0)
  o_ref[...] = jnp.dot(h, wl_ref[...], preferred_element_type=jnp.float32) \
      + bl_ref[...]


def kernel(x, edge_index, W1, b1, W2, b2, Wl, bl):
  n, d = x.shape
  h = W1.shape[1]
  e = edge_index.shape[1]

  cpt0, cpt1 = _splits(e)
  nchunks = NS * (cpt0 + cpt1)
  e_pad = nchunks * K
  n_pad = ((n + 1 + NS * ZR - 1) // (NS * ZR)) * (NS * ZR)

  pad_i = jnp.arange(e_pad - e, dtype=jnp.int32)
  src = jnp.concatenate(
      [edge_index[0], pad_i % n]).reshape(nchunks, K)
  dst = jnp.concatenate(
      [edge_index[1], n + pad_i % (n_pad - n)]).reshape(nchunks, K)

  xp = jnp.pad(x, ((0, n_pad - n), (0, 0)))
  b1r = b1.reshape(1, h)
  b2r = b2.reshape(1, h)
  blr = bl.reshape(1, 1)

  deg_kernel = _make_deg_kernel(n_pad, cpt0, cpt1)
  agg_kernel = _make_agg_kernel(n_pad, cpt0, cpt1, h)

  degp = deg_kernel(dst)

  bn = 1024
  grid = (n_pad // bn,)
  degp_spec = pl.BlockSpec((NW, bn), lambda i: (0, i))
  row_spec = pl.BlockSpec((bn, d), lambda i: (i, 0))
  p_spec = pl.BlockSpec((NC, bn, h), lambda i: (0, i, 0))
  w_spec = pl.BlockSpec((d, h), lambda i: (0, 0))
  b_spec = pl.BlockSpec((1, h), lambda i: (0, 0))

  y1 = pl.pallas_call(
      _tc1_body,
      grid=grid,
      in_specs=[degp_spec, row_spec, w_spec],
      out_specs=pl.BlockSpec((bn, h), lambda i: (i, 0)),
      out_shape=jax.ShapeDtypeStruct((n_pad, h), jnp.float32),
  )(degp, xp, W1)

  p1 = agg_kernel(y1, src, dst)

  y2 = pl.pallas_call(
      _tc2_body,
      grid=grid,
      in_specs=[degp_spec, p_spec, row_spec, b_spec, w_spec],
      out_specs=pl.BlockSpec((bn, h), lambda i: (i, 0)),
      out_shape=jax.ShapeDtypeStruct((n_pad, h), jnp.float32),
  )(degp, p1, y1, b1r, W2)

  p2 = agg_kernel(y2, src, dst)

  out = pl.pallas_call(
      _tc3_body,
      grid=grid,
      in_specs=[degp_spec, p_spec, row_spec, b_spec,
                pl.BlockSpec((h, 1), lambda i: (0, 0)),
                pl.BlockSpec((1, 1), lambda i: (0, 0))],
      out_specs=pl.BlockSpec((bn, 1), lambda i: (i, 0)),
      out_shape=jax.ShapeDtypeStruct((n_pad, 1), jnp.float32),
  )(degp, p2, y2, b2r, Wl, blr)

  return out[:n]


# speedup vs baseline: 2.4852x; 1.2169x over previous
import dataclasses
import functools

import jax
import jax.numpy as jnp
from jax import lax
from jax.experimental import pallas as pl
from jax.experimental.pallas import tpu as pltpu
from jax.experimental.pallas import tpu_sc as plsc

NC = 2
NS = 16
NW = NC * NS
LANES = 16
K = 128
ZR = 16
SG = 8
F0 = 0.5


def _splits(e):
  ctot = -(-e // (NS * K))
  ctot = ((ctot + 2 * SG - 1) // (2 * SG)) * (2 * SG)
  cpt0 = int(round(F0 * ctot / SG)) * SG
  cpt0 = min(max(cpt0, SG), ctot - SG)
  return cpt0, ctot - cpt0


def _sc_mesh():
  return plsc.VectorSubcoreMesh(core_axis_name="c", subcore_axis_name="s")


def _sc_params():
  cp = pltpu.CompilerParams()
  if "needs_layout_passes" in pltpu.CompilerParams.__dataclass_fields__:
    cp = dataclasses.replace(cp, needs_layout_passes=False)
  return cp


def _core_base(c_static, s, cpt0, cpt1):
  if c_static == 0:
    return s * cpt0
  return NS * cpt0 + s * cpt1


def _make_deg_kernel(n_pad, cpt0, cpt1):
  cptm = max(cpt0, cpt1)

  @functools.partial(
      pl.kernel,
      out_type=jax.ShapeDtypeStruct((NW, n_pad), jnp.float32),
      mesh=_sc_mesh(),
      scratch_types=[
          pltpu.VMEM((n_pad,), jnp.float32),
          pltpu.VMEM((cptm, K), jnp.int32),
      ],
      compiler_params=_sc_params(),
  )
  def deg_kernel(dst_hbm, out_hbm, deg_v, idx_v):
    c = lax.axis_index("c")
    s = lax.axis_index("s")
    wid = c * NS + s
    zeros16 = jnp.zeros((LANES,), jnp.float32)
    ones16 = jnp.ones((LANES,), jnp.float32)

    @pl.loop(0, n_pad // LANES)
    def _(i):
      deg_v[pl.ds(i * LANES, LANES)] = zeros16

    def core_body(cc, cpt):
      base = _core_base(cc, s, cpt0, cpt1)
      pltpu.sync_copy(dst_hbm.at[pl.ds(base, cpt)], idx_v.at[pl.ds(0, cpt)])

      @pl.loop(0, cpt)
      def _(j):
        for q in range(K // LANES):
          idx = idx_v[j, pl.ds(q * LANES, LANES)]
          plsc.addupdate_scatter(deg_v, [idx], ones16)

    @pl.when(c == 0)
    def _():
      core_body(0, cpt0)

    @pl.when(c == 1)
    def _():
      core_body(1, cpt1)

    pltpu.sync_copy(deg_v, out_hbm.at[wid])

  return deg_kernel


def _make_agg_kernel(n_pad, cpt0, cpt1, dh):
  rps = n_pad // NS
  cptm = max(cpt0, cpt1)

  @functools.partial(
      pl.kernel,
      out_type=jax.ShapeDtypeStruct((NC, n_pad, dh), jnp.float32),
      mesh=_sc_mesh(),
      scratch_types=[
          pltpu.VMEM((cptm, K), jnp.int32),
          pltpu.VMEM((cptm, K), jnp.int32),
          pltpu.VMEM((K, dh), jnp.float32),
          pltpu.VMEM((ZR, dh), jnp.float32),
          pltpu.VMEM_SHARED((n_pad, dh), jnp.float32),
          pltpu.SemaphoreType.DMA,
      ],
      compiler_params=_sc_params(),
  )
  def agg_kernel(y_hbm, src_hbm, dst_hbm, out_hbm,
                 src_v, dst_v, rows_v, zero_v, accum_sh, gsem):
    c = lax.axis_index("c")
    s = lax.axis_index("s")
    zeros16 = jnp.zeros((LANES,), jnp.float32)

    @pl.loop(0, ZR)
    def _(r):
      for q in range(dh // LANES):
        zero_v[r, pl.ds(q * LANES, LANES)] = zeros16

    @pl.loop(0, rps // ZR)
    def _(z):
      pltpu.sync_copy(zero_v, accum_sh.at[pl.ds(s * rps + z * ZR, ZR)])

    plsc.subcore_barrier()

    def core_body(cc, cpt):
      base = _core_base(cc, s, cpt0, cpt1)
      pltpu.sync_copy(src_hbm.at[pl.ds(base, cpt)], src_v.at[pl.ds(0, cpt)])
      pltpu.sync_copy(dst_hbm.at[pl.ds(base, cpt)], dst_v.at[pl.ds(0, cpt)])

      @pl.loop(0, cpt)
      def _(j):
        pltpu.async_copy(y_hbm.at[src_v.at[j]], rows_v, gsem).wait()
        pltpu.sync_copy(rows_v, accum_sh.at[dst_v.at[j]], add=True)

    @pl.when(c == 0)
    def _():
      core_body(0, cpt0)

    @pl.when(c == 1)
    def _():
      core_body(1, cpt1)

    plsc.subcore_barrier()

    pltpu.sync_copy(accum_sh.at[pl.ds(s * rps, rps)],
                    out_hbm.at[c].at[pl.ds(s * rps, rps)])

  return agg_kernel


def _dinv_of(degp_block):
  deg = jnp.sum(degp_block, axis=0) + 1.0
  return lax.rsqrt(deg)


def _tc1_body(degp_ref, x_ref, w_ref, y_ref):
  dinv = _dinv_of(degp_ref[...])
  xw = jnp.dot(x_ref[...], w_ref[...], preferred_element_type=jnp.float32)
  y_ref[...] = xw * dinv[:, None]


def _tc2_body(degp_ref, p_ref, y_ref, b_ref, w_ref, o_ref):
  dinv = _dinv_of(degp_ref[...])
  pre = (p_ref[0] + p_ref[1] + y_ref[...]) * dinv[:, None] + b_ref[...]
  h = jnp.maximum(pre, 0.0)
  o_ref[...] = jnp.dot(h, w_ref[...], preferred_element_type=jnp.float32) \
      * dinv[:, None]


def _tc3_body(degp_ref, p_ref, y_ref, b_ref, wl_ref, bl_ref, o_ref):
  dinv = _dinv_of(degp_ref[...])
  pre = (p_ref[0] + p_ref[1] + y_ref[...]) * dinv[:, None] + b_ref[...]
  h = jnp.maximum(pre, 0.0)
  o_ref[...] = jnp.dot(h, wl_ref[...], preferred_element_type=jnp.float32) \
      + bl_ref[...]


def kernel(x, edge_index, W1, b1, W2, b2, Wl, bl):
  n, d = x.shape
  h = W1.shape[1]
  e = edge_index.shape[1]

  cpt0, cpt1 = _splits(e)
  nchunks = NS * (cpt0 + cpt1)
  e_pad = nchunks * K
  n_pad = ((n + 1 + NS * ZR - 1) // (NS * ZR)) * (NS * ZR)

  pad_i = jnp.arange(e_pad - e, dtype=jnp.int32)
  src = jnp.concatenate(
      [edge_index[0], pad_i % n]).reshape(nchunks, K)
  dst = jnp.concatenate(
      [edge_index[1], n + pad_i % (n_pad - n)]).reshape(nchunks, K)

  xp = jnp.pad(x, ((0, n_pad - n), (0, 0)))
  b1r = b1.reshape(1, h)
  b2r = b2.reshape(1, h)
  blr = bl.reshape(1, 1)

  deg_kernel = _make_deg_kernel(n_pad, cpt0, cpt1)
  agg_kernel = _make_agg_kernel(n_pad, cpt0, cpt1, h)

  degp = deg_kernel(dst)

  bn = 1024
  grid = (n_pad // bn,)
  degp_spec = pl.BlockSpec((NW, bn), lambda i: (0, i))
  row_spec = pl.BlockSpec((bn, d), lambda i: (i, 0))
  p_spec = pl.BlockSpec((NC, bn, h), lambda i: (0, i, 0))
  w_spec = pl.BlockSpec((d, h), lambda i: (0, 0))
  b_spec = pl.BlockSpec((1, h), lambda i: (0, 0))

  y1 = pl.pallas_call(
      _tc1_body,
      grid=grid,
      in_specs=[degp_spec, row_spec, w_spec],
      out_specs=pl.BlockSpec((bn, h), lambda i: (i, 0)),
      out_shape=jax.ShapeDtypeStruct((n_pad, h), jnp.float32),
  )(degp, xp, W1)

  p1 = agg_kernel(y1, src, dst)

  y2 = pl.pallas_call(
      _tc2_body,
      grid=grid,
      in_specs=[degp_spec, p_spec, row_spec, b_spec, w_spec],
      out_specs=pl.BlockSpec((bn, h), lambda i: (i, 0)),
      out_shape=jax.ShapeDtypeStruct((n_pad, h), jnp.float32),
  )(degp, p1, y1, b1r, W2)

  p2 = agg_kernel(y2, src, dst)

  out = pl.pallas_call(
      _tc3_body,
      grid=grid,
      in_specs=[degp_spec, p_spec, row_spec, b_spec,
                pl.BlockSpec((h, 1), lambda i: (0, 0)),
                pl.BlockSpec((1, 1), lambda i: (0, 0))],
      out_specs=pl.BlockSpec((bn, 1), lambda i: (i, 0)),
      out_shape=jax.ShapeDtypeStruct((n_pad, 1), jnp.float32),
  )(degp, p2, y2, b2r, Wl, blr)

  return out[:n]


# speedup vs baseline: 3.0354x; 1.2214x over previous
import dataclasses
import functools

import jax
import jax.numpy as jnp
from jax import lax
from jax.experimental import pallas as pl
from jax.experimental.pallas import tpu as pltpu
from jax.experimental.pallas import tpu_sc as plsc

NC = 2
NS = 16
NW = NC * NS
LANES = 16
K = 128
ZR = 16
SG = 16
F0 = 0.5


def _splits(e):
  ctot = -(-e // (NS * K))
  ctot = ((ctot + 2 * SG - 1) // (2 * SG)) * (2 * SG)
  cpt0 = int(round(F0 * ctot / SG)) * SG
  cpt0 = min(max(cpt0, SG), ctot - SG)
  return cpt0, ctot - cpt0


def _sc_mesh():
  return plsc.VectorSubcoreMesh(core_axis_name="c", subcore_axis_name="s")


def _sc_params():
  cp = pltpu.CompilerParams()
  if "needs_layout_passes" in pltpu.CompilerParams.__dataclass_fields__:
    cp = dataclasses.replace(cp, needs_layout_passes=False)
  return cp


def _core_base(c_static, s, cpt0, cpt1):
  if c_static == 0:
    return s * cpt0
  return NS * cpt0 + s * cpt1


def _make_deg_kernel(n_pad, cpt0, cpt1):
  cptm = max(cpt0, cpt1)

  @functools.partial(
      pl.kernel,
      out_type=jax.ShapeDtypeStruct((NW, n_pad), jnp.float32),
      mesh=_sc_mesh(),
      scratch_types=[
          pltpu.VMEM((n_pad,), jnp.float32),
          pltpu.VMEM((cptm, K), jnp.int32),
      ],
      compiler_params=_sc_params(),
  )
  def deg_kernel(dst_hbm, out_hbm, deg_v, idx_v):
    c = lax.axis_index("c")
    s = lax.axis_index("s")
    wid = c * NS + s
    zeros16 = jnp.zeros((LANES,), jnp.float32)
    ones16 = jnp.ones((LANES,), jnp.float32)

    @pl.loop(0, n_pad // LANES)
    def _(i):
      deg_v[pl.ds(i * LANES, LANES)] = zeros16

    def core_body(cc, cpt):
      base = _core_base(cc, s, cpt0, cpt1)
      pltpu.sync_copy(dst_hbm.at[pl.ds(base, cpt)], idx_v.at[pl.ds(0, cpt)])

      @pl.loop(0, cpt)
      def _(j):
        for q in range(K // LANES):
          idx = idx_v[j, pl.ds(q * LANES, LANES)]
          plsc.addupdate_scatter(deg_v, [idx], ones16)

    @pl.when(c == 0)
    def _():
      core_body(0, cpt0)

    @pl.when(c == 1)
    def _():
      core_body(1, cpt1)

    pltpu.sync_copy(deg_v, out_hbm.at[wid])

  return deg_kernel


def _make_agg_kernel(n_pad, cpt0, cpt1, dh):
  rps = n_pad // NS

  @functools.partial(
      pl.kernel,
      out_type=jax.ShapeDtypeStruct((NC, n_pad, dh), jnp.float32),
      mesh=_sc_mesh(),
      scratch_types=[
          pltpu.VMEM((SG, K), jnp.int32),
          pltpu.VMEM((SG, K), jnp.int32),
          pltpu.VMEM((K, dh), jnp.float32),
          pltpu.VMEM((K, dh), jnp.float32),
          pltpu.VMEM((ZR, dh), jnp.float32),
          pltpu.VMEM_SHARED((n_pad, dh), jnp.float32),
          pltpu.SemaphoreType.DMA,
          pltpu.SemaphoreType.DMA,
      ],
      compiler_params=_sc_params(),
  )
  def agg_kernel(y_hbm, src_hbm, dst_hbm, out_hbm,
                 src_v, dst_v, rows0_v, rows1_v, zero_v, accum_sh,
                 gsem0, gsem1):
    c = lax.axis_index("c")
    s = lax.axis_index("s")
    zeros16 = jnp.zeros((LANES,), jnp.float32)

    def start_g(j, rows_v, sem):
      pltpu.make_async_copy(y_hbm.at[src_v.at[j]], rows_v, sem).start()

    def wait_g(rows_v, sem):
      pltpu.make_async_copy(y_hbm.at[src_v.at[0]], rows_v, sem).wait()

    def scat(j, rows_v):
      pltpu.sync_copy(rows_v, accum_sh.at[dst_v.at[j]], add=True)

    @pl.loop(0, ZR)
    def _(r):
      for q in range(dh // LANES):
        zero_v[r, pl.ds(q * LANES, LANES)] = zeros16

    @pl.loop(0, rps // ZR)
    def _(z):
      pltpu.sync_copy(zero_v, accum_sh.at[pl.ds(s * rps + z * ZR, ZR)])

    plsc.subcore_barrier()

    def core_body(cc, cpt):
      base = _core_base(cc, s, cpt0, cpt1)

      @pl.loop(0, cpt // SG)
      def _(st):
        pltpu.sync_copy(src_hbm.at[pl.ds(base + st * SG, SG)], src_v)
        pltpu.sync_copy(dst_hbm.at[pl.ds(base + st * SG, SG)], dst_v)
        start_g(0, rows0_v, gsem0)

        @pl.loop(0, SG - 2, step=2)
        def _(jj):
          wait_g(rows0_v, gsem0)
          start_g(jj + 1, rows1_v, gsem1)
          scat(jj, rows0_v)
          wait_g(rows1_v, gsem1)
          start_g(jj + 2, rows0_v, gsem0)
          scat(jj + 1, rows1_v)

        wait_g(rows0_v, gsem0)
        start_g(SG - 1, rows1_v, gsem1)
        scat(SG - 2, rows0_v)
        wait_g(rows1_v, gsem1)
        scat(SG - 1, rows1_v)

    @pl.when(c == 0)
    def _():
      core_body(0, cpt0)

    @pl.when(c == 1)
    def _():
      core_body(1, cpt1)

    plsc.subcore_barrier()

    pltpu.sync_copy(accum_sh.at[pl.ds(s * rps, rps)],
                    out_hbm.at[c].at[pl.ds(s * rps, rps)])

  return agg_kernel


def _dinv_of(degp_block):
  deg = jnp.sum(degp_block, axis=0) + 1.0
  return lax.rsqrt(deg)


def _tc1_body(degp_ref, x_ref, w_ref, y_ref):
  dinv = _dinv_of(degp_ref[...])
  xw = jnp.dot(x_ref[...], w_ref[...], preferred_element_type=jnp.float32)
  y_ref[...] = xw * dinv[:, None]


def _tc2_body(degp_ref, p_ref, y_ref, b_ref, w_ref, o_ref):
  dinv = _dinv_of(degp_ref[...])
  pre = (p_ref[0] + p_ref[1] + y_ref[...]) * dinv[:, None] + b_ref[...]
  h = jnp.maximum(pre, 0.0)
  o_ref[...] = jnp.dot(h, w_ref[...], preferred_element_type=jnp.float32) \
      * dinv[:, None]


def _tc3_body(degp_ref, p_ref, y_ref, b_ref, wl_ref, bl_ref, o_ref):
  dinv = _dinv_of(degp_ref[...])
  pre = (p_ref[0] + p_ref[1] + y_ref[...]) * dinv[:, None] + b_ref[...]
  h = jnp.maximum(pre, 0.0)
  o_ref[...] = jnp.dot(h, wl_ref[...], preferred_element_type=jnp.float32) \
      + bl_ref[...]


def kernel(x, edge_index, W1, b1, W2, b2, Wl, bl):
  n, d = x.shape
  h = W1.shape[1]
  e = edge_index.shape[1]

  cpt0, cpt1 = _splits(e)
  nchunks = NS * (cpt0 + cpt1)
  e_pad = nchunks * K
  n_pad = ((n + 1 + NS * ZR - 1) // (NS * ZR)) * (NS * ZR)

  pad_i = jnp.arange(e_pad - e, dtype=jnp.int32)
  src = jnp.concatenate(
      [edge_index[0], pad_i % n]).reshape(nchunks, K)
  dst = jnp.concatenate(
      [edge_index[1], n + pad_i % (n_pad - n)]).reshape(nchunks, K)

  xp = jnp.pad(x, ((0, n_pad - n), (0, 0)))
  b1r = b1.reshape(1, h)
  b2r = b2.reshape(1, h)
  blr = bl.reshape(1, 1)

  deg_kernel = _make_deg_kernel(n_pad, cpt0, cpt1)
  agg_kernel = _make_agg_kernel(n_pad, cpt0, cpt1, h)

  degp = deg_kernel(dst)

  bn = 1024
  grid = (n_pad // bn,)
  degp_spec = pl.BlockSpec((NW, bn), lambda i: (0, i))
  row_spec = pl.BlockSpec((bn, d), lambda i: (i, 0))
  p_spec = pl.BlockSpec((NC, bn, h), lambda i: (0, i, 0))
  w_spec = pl.BlockSpec((d, h), lambda i: (0, 0))
  b_spec = pl.BlockSpec((1, h), lambda i: (0, 0))

  y1 = pl.pallas_call(
      _tc1_body,
      grid=grid,
      in_specs=[degp_spec, row_spec, w_spec],
      out_specs=pl.BlockSpec((bn, h), lambda i: (i, 0)),
      out_shape=jax.ShapeDtypeStruct((n_pad, h), jnp.float32),
  )(degp, xp, W1)

  p1 = agg_kernel(y1, src, dst)

  y2 = pl.pallas_call(
      _tc2_body,
      grid=grid,
      in_specs=[degp_spec, p_spec, row_spec, b_spec, w_spec],
      out_specs=pl.BlockSpec((bn, h), lambda i: (i, 0)),
      out_shape=jax.ShapeDtypeStruct((n_pad, h), jnp.float32),
  )(degp, p1, y1, b1r, W2)

  p2 = agg_kernel(y2, src, dst)

  out = pl.pallas_call(
      _tc3_body,
      grid=grid,
      in_specs=[degp_spec, p_spec, row_spec, b_spec,
                pl.BlockSpec((h, 1), lambda i: (0, 0)),
                pl.BlockSpec((1, 1), lambda i: (0, 0))],
      out_specs=pl.BlockSpec((bn, 1), lambda i: (i, 0)),
      out_shape=jax.ShapeDtypeStruct((n_pad, 1), jnp.float32),
  )(degp, p2, y2, b2r, Wl, blr)

  return out[:n]
